# baseline (speedup 1.0000x reference)
import functools

import jax
import jax.numpy as jnp
from jax import lax
from jax.experimental import pallas as pl
from jax.experimental.pallas import tpu as pltpu
from jax.experimental.pallas import tpu_sc as plsc

N = 10000
D = 128
E = 320000

NC = 2
NS = 16
NW = NC * NS
CHUNK = 128
SEG = 16
CA = 80
CB = 80
TOTC = NS * (CA + CB)
EPAD = TOTC * CHUNK
NACC = 10240
RPT = NACC // NS
BM = 1000
GRID = N // BM

_MESH = dict(core_axis_name="c", subcore_axis_name="s")


def _sc_degree(dst3):
    @functools.partial(
        pl.kernel,
        out_type=jax.ShapeDtypeStruct((NC, NACC, 16), jnp.float32),
        mesh=plsc.VectorSubcoreMesh(**_MESH),
        scratch_types=[
            pltpu.VMEM((CA, CHUNK), jnp.int32),
            pltpu.VMEM((CHUNK, 16), jnp.float32),
            pltpu.VMEM_SHARED((NACC, 16), jnp.float32),
            pltpu.SemaphoreType.DMA,
        ],
    )
    def k(dst_hbm, out_hbm, didx, ones_v, acc_sh, sem):
        c = lax.axis_index("c")
        s = lax.axis_index("s")
        start = jnp.where(c == 0, s * CA, NS * CA + s * CB)
        cnt = jnp.where(c == 0, CA, CB)

        @pl.loop(0, CHUNK)
        def _(i):
            ones_v[i, :] = jnp.zeros((16,), jnp.float32)

        @pl.loop(0, RPT // CHUNK)
        def _(kb):
            pltpu.sync_copy(ones_v,
                            acc_sh.at[pl.ds(s * RPT + kb * CHUNK, CHUNK)])

        @pl.loop(0, CHUNK)
        def _(i):
            ones_v[i, :] = jnp.ones((16,), jnp.float32)

        off = pl.multiple_of(start, SEG)
        ncp = pl.multiple_of(cnt, SEG)
        pltpu.sync_copy(dst_hbm.at[pl.ds(off, ncp)], didx.at[pl.ds(0, ncp)])
        plsc.subcore_barrier()

        @pl.loop(0, cnt)
        def _(j):
            pltpu.sync_copy(ones_v, acc_sh.at[didx.at[j]], add=True)

        plsc.subcore_barrier()
        pltpu.sync_copy(acc_sh.at[pl.ds(s * RPT, RPT)],
                        out_hbm.at[c, pl.ds(s * RPT, RPT)])

    return k(dst3)


def _sc_aggregate(table, src3, dst3):
    @functools.partial(
        pl.kernel,
        out_type=jax.ShapeDtypeStruct((NC, NACC, D), jnp.float32),
        mesh=plsc.VectorSubcoreMesh(**_MESH),
        scratch_types=[
            pltpu.VMEM((CA, CHUNK), jnp.int32),
            pltpu.VMEM((CA, CHUNK), jnp.int32),
            pltpu.VMEM((CHUNK, D), jnp.float32),
            pltpu.VMEM_SHARED((NACC, D), jnp.float32),
            pltpu.SemaphoreType.DMA,
        ],
    )
    def k(table_hbm, src_hbm, dst_hbm, out_hbm, sidx, didx, rows0,
          acc_sh, sem0):
        c = lax.axis_index("c")
        s = lax.axis_index("s")
        start = jnp.where(c == 0, s * CA, NS * CA + s * CB)
        cnt = jnp.where(c == 0, CA, CB)

        @pl.loop(0, CHUNK)
        def _(i):
            @pl.loop(0, D // 16)
            def _(kk):
                rows0[i, pl.ds(kk * 16, 16)] = jnp.zeros((16,), jnp.float32)

        @pl.loop(0, RPT // CHUNK)
        def _(kb):
            pltpu.sync_copy(rows0,
                            acc_sh.at[pl.ds(s * RPT + kb * CHUNK, CHUNK)])

        off = pl.multiple_of(start, SEG)
        ncp = pl.multiple_of(cnt, SEG)
        pltpu.sync_copy(src_hbm.at[pl.ds(off, ncp)], sidx.at[pl.ds(0, ncp)])
        pltpu.sync_copy(dst_hbm.at[pl.ds(off, ncp)], didx.at[pl.ds(0, ncp)])
        plsc.subcore_barrier()

        @pl.loop(0, cnt)
        def _(j):
            pltpu.sync_copy(table_hbm.at[sidx.at[j]], rows0)
            pltpu.sync_copy(rows0, acc_sh.at[didx.at[j]], add=True)

        plsc.subcore_barrier()
        pltpu.sync_copy(acc_sh.at[pl.ds(s * RPT, RPT)],
                        out_hbm.at[c, pl.ds(s * RPT, RPT)])

    return k(table, src3, dst3)


def _mm_body(x_ref, w_ref, o_ref):
    o_ref[...] = jnp.dot(x_ref[...], w_ref[...],
                         preferred_element_type=jnp.float32)


def _tc_matmul(x, W):
    return pl.pallas_call(
        _mm_body,
        grid=(GRID,),
        in_specs=[pl.BlockSpec((BM, D), lambda i: (i, 0)),
                  pl.BlockSpec((D, D), lambda i: (0, 0))],
        out_specs=pl.BlockSpec((BM, D), lambda i: (i, 0)),
        out_shape=jax.ShapeDtypeStruct((N, D), jnp.float32),
    )(x, W)


def _u_of(deg_ref):
    deg = deg_ref[0, :, 0] + deg_ref[1, :, 0] + 1.0
    return lax.rsqrt(deg)


def _scale_body(h_ref, deg_ref, o_ref):
    o_ref[...] = h_ref[...] * _u_of(deg_ref)[:, None]


def _tc_scale(h, degp):
    return pl.pallas_call(
        _scale_body,
        grid=(GRID,),
        in_specs=[pl.BlockSpec((BM, D), lambda i: (i, 0)),
                  pl.BlockSpec((NC, BM, 16), lambda i: (0, i, 0))],
        out_specs=pl.BlockSpec((BM, D), lambda i: (i, 0)),
        out_shape=jax.ShapeDtypeStruct((N, D), jnp.float32),
    )(h, degp)


def _mid_body(agg_ref, hp_ref, deg_ref, b1_ref, w2_ref, o_ref):
    u = _u_of(deg_ref)
    t = (agg_ref[0] + agg_ref[1] + hp_ref[...]) * u[:, None] + b1_ref[...]
    z = jnp.maximum(t, 0.0)
    o_ref[...] = jnp.dot(z, w2_ref[...], preferred_element_type=jnp.float32)


def _tc_mid(agg1, hp, degp, b1, W2):
    return pl.pallas_call(
        _mid_body,
        grid=(GRID,),
        in_specs=[pl.BlockSpec((NC, BM, D), lambda i: (0, i, 0)),
                  pl.BlockSpec((BM, D), lambda i: (i, 0)),
                  pl.BlockSpec((NC, BM, 16), lambda i: (0, i, 0)),
                  pl.BlockSpec((1, D), lambda i: (0, 0)),
                  pl.BlockSpec((D, D), lambda i: (0, 0))],
        out_specs=pl.BlockSpec((BM, D), lambda i: (i, 0)),
        out_shape=jax.ShapeDtypeStruct((N, D), jnp.float32),
    )(agg1, hp, degp, b1, W2)


def _final_body(agg_ref, b2_ref, o_ref):
    o_ref[...] = agg_ref[0] + agg_ref[1] + b2_ref[...]


def _tc_final(agg2, b2):
    return pl.pallas_call(
        _final_body,
        grid=(GRID,),
        in_specs=[pl.BlockSpec((NC, BM, D), lambda i: (0, i, 0)),
                  pl.BlockSpec((1, D), lambda i: (0, 0))],
        out_specs=pl.BlockSpec((BM, D), lambda i: (i, 0)),
        out_shape=jax.ShapeDtypeStruct((N, D), jnp.float32),
    )(agg2, b2)


def kernel(x, edge_index, W1, b1, W2, b2):
    src = edge_index[0].astype(jnp.int32)
    dst = edge_index[1].astype(jnp.int32)
    pad = EPAD - E
    src3 = jnp.concatenate([src, jnp.zeros((pad,), jnp.int32)]).reshape(TOTC, CHUNK)
    dst3 = jnp.concatenate([dst, jnp.full((pad,), N, jnp.int32)]).reshape(TOTC, CHUNK)

    degp = _sc_degree(dst3)
    h = _tc_matmul(x, W1)
    hp = _tc_scale(h, degp)
    agg1 = _sc_aggregate(hp, src3, dst3)
    h2 = _tc_mid(agg1, hp, degp, b1.reshape(1, D), W2)
    agg2 = _sc_aggregate(h2, src3, dst3)
    return _tc_final(agg2, b2.reshape(1, D))


# speedup vs baseline: 1.5730x; 1.5730x over previous
import functools

import jax
import jax.numpy as jnp
from jax import lax
from jax.experimental import pallas as pl
from jax.experimental.pallas import tpu as pltpu
from jax.experimental.pallas import tpu_sc as plsc

N = 10000
D = 128
E = 320000

NC = 2
NS = 16
NW = NC * NS
CHUNK = 128
CPT = -(-(E // NW) // CHUNK)
EPT = CPT * CHUNK
NACC = 10240
RPT = NACC // NS
BM = 1000
GRID = N // BM

_MESH = dict(core_axis_name="c", subcore_axis_name="s")


def _sc_degree(dst3):
    @functools.partial(
        pl.kernel,
        out_type=jax.ShapeDtypeStruct((NC, NACC, 16), jnp.float32),
        mesh=plsc.VectorSubcoreMesh(**_MESH),
        scratch_types=[
            pltpu.VMEM((CPT, CHUNK), jnp.int32),
            pltpu.VMEM((CHUNK, 16), jnp.float32),
            pltpu.VMEM_SHARED((NACC, 16), jnp.float32),
            pltpu.SemaphoreType.DMA,
        ],
    )
    def k(dst_hbm, out_hbm, idx_v, ones_v, acc_sh, sem):
        c = lax.axis_index("c")
        s = lax.axis_index("s")
        wid = c * NS + s

        @pl.loop(0, CHUNK)
        def _(i):
            ones_v[i, :] = jnp.zeros((16,), jnp.float32)

        @pl.loop(0, RPT // CHUNK)
        def _(kb):
            pltpu.sync_copy(ones_v, acc_sh.at[pl.ds(s * RPT + kb * CHUNK, CHUNK)])

        @pl.loop(0, CHUNK)
        def _(i):
            ones_v[i, :] = jnp.ones((16,), jnp.float32)

        pltpu.sync_copy(dst_hbm.at[wid], idx_v)
        plsc.subcore_barrier()

        @pl.loop(0, CPT)
        def _(j):
            pltpu.sync_copy(ones_v, acc_sh.at[idx_v.at[j]], add=True)

        plsc.subcore_barrier()
        pltpu.sync_copy(acc_sh.at[pl.ds(s * RPT, RPT)],
                        out_hbm.at[c, pl.ds(s * RPT, RPT)])

    return k(dst3)


def _sc_aggregate(table, src3, dst3):
    @functools.partial(
        pl.kernel,
        out_type=jax.ShapeDtypeStruct((NC, NACC, D), jnp.float32),
        mesh=plsc.VectorSubcoreMesh(**_MESH),
        scratch_types=[
            pltpu.VMEM((CPT, CHUNK), jnp.int32),
            pltpu.VMEM((CPT, CHUNK), jnp.int32),
            pltpu.VMEM((CHUNK, D), jnp.float32),
            pltpu.VMEM_SHARED((NACC, D), jnp.float32),
            pltpu.SemaphoreType.DMA,
        ],
    )
    def k(table_hbm, src_hbm, dst_hbm, out_hbm, sidx, didx, rows, acc_sh, sem):
        c = lax.axis_index("c")
        s = lax.axis_index("s")
        wid = c * NS + s

        @pl.loop(0, CHUNK)
        def _(i):
            @pl.loop(0, D // 16)
            def _(kk):
                rows[i, pl.ds(kk * 16, 16)] = jnp.zeros((16,), jnp.float32)

        @pl.loop(0, RPT // CHUNK)
        def _(kb):
            pltpu.sync_copy(rows, acc_sh.at[pl.ds(s * RPT + kb * CHUNK, CHUNK)])

        pltpu.sync_copy(src_hbm.at[wid], sidx)
        pltpu.sync_copy(dst_hbm.at[wid], didx)
        plsc.subcore_barrier()

        @pl.loop(0, CPT)
        def _(j):
            pltpu.sync_copy(table_hbm.at[sidx.at[j]], rows)
            pltpu.sync_copy(rows, acc_sh.at[didx.at[j]], add=True)

        plsc.subcore_barrier()
        pltpu.sync_copy(acc_sh.at[pl.ds(s * RPT, RPT)],
                        out_hbm.at[c, pl.ds(s * RPT, RPT)])

    return k(table, src3, dst3)


def _mm_body(x_ref, w_ref, o_ref):
    o_ref[...] = jnp.dot(x_ref[...], w_ref[...],
                         preferred_element_type=jnp.float32)


def _tc_matmul(x, W):
    return pl.pallas_call(
        _mm_body,
        grid=(GRID,),
        in_specs=[pl.BlockSpec((BM, D), lambda i: (i, 0)),
                  pl.BlockSpec((D, D), lambda i: (0, 0))],
        out_specs=pl.BlockSpec((BM, D), lambda i: (i, 0)),
        out_shape=jax.ShapeDtypeStruct((N, D), jnp.float32),
    )(x, W)


def _u_of(deg_ref):
    deg = deg_ref[0, :, 0] + deg_ref[1, :, 0] + 1.0
    return lax.rsqrt(deg)


def _scale_body(h_ref, deg_ref, o_ref):
    o_ref[...] = h_ref[...] * _u_of(deg_ref)[:, None]


def _tc_scale(h, degp):
    return pl.pallas_call(
        _scale_body,
        grid=(GRID,),
        in_specs=[pl.BlockSpec((BM, D), lambda i: (i, 0)),
                  pl.BlockSpec((NC, BM, 16), lambda i: (0, i, 0))],
        out_specs=pl.BlockSpec((BM, D), lambda i: (i, 0)),
        out_shape=jax.ShapeDtypeStruct((N, D), jnp.float32),
    )(h, degp)


def _mid_body(agg_ref, hp_ref, deg_ref, b1_ref, w2_ref, o_ref):
    u = _u_of(deg_ref)
    t = (agg_ref[0] + agg_ref[1] + hp_ref[...]) * u[:, None] + b1_ref[...]
    z = jnp.maximum(t, 0.0)
    o_ref[...] = jnp.dot(z, w2_ref[...], preferred_element_type=jnp.float32)


def _tc_mid(agg1, hp, degp, b1, W2):
    return pl.pallas_call(
        _mid_body,
        grid=(GRID,),
        in_specs=[pl.BlockSpec((NC, BM, D), lambda i: (0, i, 0)),
                  pl.BlockSpec((BM, D), lambda i: (i, 0)),
                  pl.BlockSpec((NC, BM, 16), lambda i: (0, i, 0)),
                  pl.BlockSpec((1, D), lambda i: (0, 0)),
                  pl.BlockSpec((D, D), lambda i: (0, 0))],
        out_specs=pl.BlockSpec((BM, D), lambda i: (i, 0)),
        out_shape=jax.ShapeDtypeStruct((N, D), jnp.float32),
    )(agg1, hp, degp, b1, W2)


def _final_body(agg_ref, b2_ref, o_ref):
    o_ref[...] = agg_ref[0] + agg_ref[1] + b2_ref[...]


def _tc_final(agg2, b2):
    return pl.pallas_call(
        _final_body,
        grid=(GRID,),
        in_specs=[pl.BlockSpec((NC, BM, D), lambda i: (0, i, 0)),
                  pl.BlockSpec((1, D), lambda i: (0, 0))],
        out_specs=pl.BlockSpec((BM, D), lambda i: (i, 0)),
        out_shape=jax.ShapeDtypeStruct((N, D), jnp.float32),
    )(agg2, b2)


def kernel(x, edge_index, W1, b1, W2, b2):
    src = edge_index[0].astype(jnp.int32)
    dst = edge_index[1].astype(jnp.int32)
    pad = NW * EPT - E
    src3 = jnp.concatenate([src, jnp.zeros((pad,), jnp.int32)]).reshape(NW, CPT, CHUNK)
    dst3 = jnp.concatenate([dst, jnp.full((pad,), N, jnp.int32)]).reshape(NW, CPT, CHUNK)

    degp = _sc_degree(dst3)
    h = _tc_matmul(x, W1)
    hp = _tc_scale(h, degp)
    agg1 = _sc_aggregate(hp, src3, dst3)
    h2 = _tc_mid(agg1, hp, degp, b1.reshape(1, D), W2)
    agg2 = _sc_aggregate(h2, src3, dst3)
    return _tc_final(agg2, b2.reshape(1, D))
